# baseline (speedup 1.0000x reference)
import functools

import jax
import jax.numpy as jnp
from jax import lax
from jax.experimental import pallas as pl
from jax.experimental.pallas import tpu as pltpu
from jax.experimental.pallas import tpu_sc as plsc

_LANES = 16
_NWORKERS = 32
_K = 64
_UNROLL = 8


def _bilinear_sc(table, gx, gy, npix, h, w, c):
    ppw = npix // _NWORKERS
    nchunk = ppw // _K
    npairs = nchunk // 2
    cvecs = c // _LANES

    mesh = plsc.VectorSubcoreMesh(core_axis_name="c", subcore_axis_name="s")

    buf_types = (
        [pltpu.VMEM((_K,), jnp.int32)] * 4
        + [pltpu.VMEM((_K,), jnp.float32)] * 4
        + [pltpu.VMEM((_K, c), jnp.float32)] * 4
    )

    @functools.partial(
        pl.kernel,
        out_type=jax.ShapeDtypeStruct((npix, c), jnp.float32),
        mesh=mesh,
        compiler_params=pltpu.CompilerParams(
            needs_layout_passes=False, use_tc_tiling_on_sc=False),
        scratch_types=(
            [pltpu.VMEM((_K,), jnp.float32)] * 2
            + buf_types
            + buf_types
            + [pltpu.VMEM((_K, c), jnp.float32)] * 2
            + [pltpu.SemaphoreType.DMA] * 4
        ),
    )
    def kern(table_hbm, gx_hbm, gy_hbm, out_hbm, *s):
        gxv, gyv = s[0], s[1]
        ia, wa, ra = s[2:6], s[6:10], s[10:14]
        ib, wb, rb = s[14:18], s[18:22], s[22:26]
        outa, outb = s[26], s[27]
        sema, semb, semoa, semob = s[28], s[29], s[30], s[31]

        wid = lax.axis_index("s") * 2 + lax.axis_index("c")
        tbase = wid * ppw

        def lc(ci, iv, wv):
            base = tbase + ci * _K
            pltpu.sync_copy(gx_hbm.at[pl.ds(base, _K)], gxv)
            pltpu.sync_copy(gy_hbm.at[pl.ds(base, _K)], gyv)
            one = jnp.float32(1.0)
            zero = jnp.float32(0.0)
            for g in range(_K // _LANES):
                sl = pl.ds(g * _LANES, _LANES)
                x = gxv[sl] * jnp.float32(w * 0.5) + jnp.float32(w * 0.5 - 0.5)
                y = gyv[sl] * jnp.float32(h * 0.5) + jnp.float32(h * 0.5 - 0.5)
                x0 = x.astype(jnp.int32)
                x0 = jnp.where(x0.astype(jnp.float32) > x, x0 - 1, x0)
                y0 = y.astype(jnp.int32)
                y0 = jnp.where(y0.astype(jnp.float32) > y, y0 - 1, y0)
                fx = x - x0.astype(jnp.float32)
                fy = y - y0.astype(jnp.float32)
                x1 = x0 + 1
                y1 = y0 + 1
                vx0 = jnp.where((x0 >= 0) & (x0 <= w - 1), one, zero)
                vx1 = jnp.where((x1 >= 0) & (x1 <= w - 1), one, zero)
                vy0 = jnp.where((y0 >= 0) & (y0 <= h - 1), one, zero)
                vy1 = jnp.where((y1 >= 0) & (y1 <= h - 1), one, zero)
                cx0 = jnp.clip(x0, 0, w - 1)
                cx1 = jnp.clip(x1, 0, w - 1)
                ry0 = jnp.clip(y0, 0, h - 1) * w
                ry1 = jnp.clip(y1, 0, h - 1) * w
                iv[0][sl] = ry0 + cx0
                iv[1][sl] = ry0 + cx1
                iv[2][sl] = ry1 + cx0
                iv[3][sl] = ry1 + cx1
                wv[0][sl] = (one - fx) * vx0
                wv[1][sl] = fx * vx1
                wv[2][sl] = (one - fy) * vy0
                wv[3][sl] = fy * vy1

        def launch(iv, rv, sem):
            for t in range(4):
                pltpu.async_copy(table_hbm.at[iv[t]], rv[t], sem)

        def wait4(iv, rv, sem):
            for t in range(4):
                pltpu.make_async_copy(table_hbm.at[iv[t]], rv[t], sem).wait()

        def combine(wv, rv, outv):
            def px_body(kk, carry2):
                k0 = kk * _UNROLL
                for u in range(_UNROLL):
                    k = k0 + u
                    kb = jnp.full((_LANES,), k, jnp.int32)
                    bx0 = plsc.load_gather(wv[0], [kb])
                    bx1 = plsc.load_gather(wv[1], [kb])
                    by0 = plsc.load_gather(wv[2], [kb])
                    by1 = plsc.load_gather(wv[3], [kb])
                    for j in range(cvecs):
                        cs = pl.ds(j * _LANES, _LANES)
                        t0 = rv[0][k, cs] * bx0 + rv[1][k, cs] * bx1
                        t1 = rv[2][k, cs] * bx0 + rv[3][k, cs] * bx1
                        outv[k, cs] = t0 * by0 + t1 * by1
                return carry2

            lax.fori_loop(0, _K // _UNROLL, px_body, 0)

        def awrite(ci, outv, sem):
            pltpu.async_copy(outv, out_hbm.at[pl.ds(tbase + ci * _K, _K)], sem)

        def waitout(outv, sem):
            pltpu.make_async_copy(out_hbm.at[pl.ds(tbase, _K)], outv, sem).wait()

        lc(0, ia, wa)
        launch(ia, ra, sema)
        lc(1, ib, wb)
        launch(ib, rb, semb)
        wait4(ia, ra, sema)
        combine(wa, ra, outa)
        awrite(0, outa, semoa)
        lc(2, ia, wa)
        launch(ia, ra, sema)
        wait4(ib, rb, semb)
        combine(wb, rb, outb)
        awrite(1, outb, semob)

        def body(p, carry):
            lc(2 * p + 1, ib, wb)
            launch(ib, rb, semb)
            wait4(ia, ra, sema)
            waitout(outa, semoa)
            combine(wa, ra, outa)
            awrite(2 * p, outa, semoa)
            lc(2 * p + 2, ia, wa)
            launch(ia, ra, sema)
            wait4(ib, rb, semb)
            waitout(outb, semob)
            combine(wb, rb, outb)
            awrite(2 * p + 1, outb, semob)
            return carry

        lax.fori_loop(1, npairs - 1, body, 0)

        lc(nchunk - 1, ib, wb)
        launch(ib, rb, semb)
        wait4(ia, ra, sema)
        waitout(outa, semoa)
        combine(wa, ra, outa)
        awrite(nchunk - 2, outa, semoa)
        wait4(ib, rb, semb)
        waitout(outb, semob)
        combine(wb, rb, outb)
        awrite(nchunk - 1, outb, semob)
        waitout(outa, semoa)
        waitout(outb, semob)

    return kern(table, gx, gy)


def kernel(inputImages, grids):
    c, h, w = inputImages.shape
    b, hg, wg, _ = grids.shape
    npix = b * hg * wg
    table = inputImages.reshape(c, h * w).T
    gx = grids[..., 0].reshape(-1)
    gy = grids[..., 1].reshape(-1)
    out_t = _bilinear_sc(table, gx, gy, npix, h, w, c)
    return out_t.reshape(b, hg, wg, c).transpose(0, 3, 1, 2)


# speedup vs baseline: 1.2560x; 1.2560x over previous
import functools

import jax
import jax.numpy as jnp
from jax import lax
from jax.experimental import pallas as pl
from jax.experimental.pallas import tpu as pltpu
from jax.experimental.pallas import tpu_sc as plsc

_LANES = 16
_NWORKERS = 32
_K = 64
_UNROLL = 4


def _bilinear_sc(table, gx, gy, npix, h, w, c):
    ppw = npix // _NWORKERS
    nchunk = ppw // _K
    npairs = nchunk // 2
    cvecs = c // _LANES

    mesh = plsc.VectorSubcoreMesh(core_axis_name="c", subcore_axis_name="s")

    buf_types = (
        [pltpu.VMEM((_K,), jnp.int32)] * 4
        + [pltpu.VMEM((_K,), jnp.float32)] * 4
        + [pltpu.VMEM((_K, c), jnp.float32)] * 4
    )

    @functools.partial(
        pl.kernel,
        out_type=jax.ShapeDtypeStruct((npix, c), jnp.float32),
        mesh=mesh,
        compiler_params=pltpu.CompilerParams(
            needs_layout_passes=False, use_tc_tiling_on_sc=False),
        scratch_types=(
            [pltpu.VMEM((ppw,), jnp.float32)] * 2
            + buf_types
            + buf_types
            + [pltpu.VMEM((_K, c), jnp.float32)] * 2
            + [pltpu.SemaphoreType.DMA] * 4
        ),
    )
    def kern(table_hbm, gx_hbm, gy_hbm, out_hbm, *s):
        gxv, gyv = s[0], s[1]
        ia, wa, ra = s[2:6], s[6:10], s[10:14]
        ib, wb, rb = s[14:18], s[18:22], s[22:26]
        outa, outb = s[26], s[27]
        sema, semb, semoa, semob = s[28], s[29], s[30], s[31]

        wid = lax.axis_index("s") * 2 + lax.axis_index("c")
        tbase = wid * ppw

        pltpu.sync_copy(gx_hbm.at[pl.ds(tbase, ppw)], gxv)
        pltpu.sync_copy(gy_hbm.at[pl.ds(tbase, ppw)], gyv)

        def lc(ci, iv, wv):
            lbase = ci * _K
            one = jnp.float32(1.0)
            zero = jnp.float32(0.0)
            for g in range(_K // _LANES):
                sl = pl.ds(g * _LANES, _LANES)
                gsl = pl.ds(lbase + g * _LANES, _LANES)
                x = gxv[gsl] * jnp.float32(w * 0.5) + jnp.float32(w * 0.5 - 0.5)
                y = gyv[gsl] * jnp.float32(h * 0.5) + jnp.float32(h * 0.5 - 0.5)
                x0 = x.astype(jnp.int32)
                x0 = jnp.where(x0.astype(jnp.float32) > x, x0 - 1, x0)
                y0 = y.astype(jnp.int32)
                y0 = jnp.where(y0.astype(jnp.float32) > y, y0 - 1, y0)
                fx = x - x0.astype(jnp.float32)
                fy = y - y0.astype(jnp.float32)
                x1 = x0 + 1
                y1 = y0 + 1
                vx0 = jnp.where((x0 >= 0) & (x0 <= w - 1), one, zero)
                vx1 = jnp.where((x1 >= 0) & (x1 <= w - 1), one, zero)
                vy0 = jnp.where((y0 >= 0) & (y0 <= h - 1), one, zero)
                vy1 = jnp.where((y1 >= 0) & (y1 <= h - 1), one, zero)
                cx0 = jnp.clip(x0, 0, w - 1)
                cx1 = jnp.clip(x1, 0, w - 1)
                ry0 = jnp.clip(y0, 0, h - 1) * w
                ry1 = jnp.clip(y1, 0, h - 1) * w
                iv[0][sl] = ry0 + cx0
                iv[1][sl] = ry0 + cx1
                iv[2][sl] = ry1 + cx0
                iv[3][sl] = ry1 + cx1
                wv[0][sl] = (one - fx) * vx0
                wv[1][sl] = fx * vx1
                wv[2][sl] = (one - fy) * vy0
                wv[3][sl] = fy * vy1

        def launch(iv, rv, sem):
            for t in range(4):
                pltpu.async_copy(table_hbm.at[iv[t]], rv[t], sem)

        def wait4(iv, rv, sem):
            for t in range(4):
                pltpu.make_async_copy(table_hbm.at[iv[t]], rv[t], sem).wait()

        def combine(wv, rv, outv):
            def px_body(kk, carry2):
                k0 = kk * _UNROLL
                for u in range(_UNROLL):
                    k = k0 + u
                    kb = jnp.full((_LANES,), k, jnp.int32)
                    bx0 = plsc.load_gather(wv[0], [kb])
                    bx1 = plsc.load_gather(wv[1], [kb])
                    by0 = plsc.load_gather(wv[2], [kb])
                    by1 = plsc.load_gather(wv[3], [kb])
                    for j in range(cvecs):
                        cs = pl.ds(j * _LANES, _LANES)
                        t0 = rv[0][k, cs] * bx0 + rv[1][k, cs] * bx1
                        t1 = rv[2][k, cs] * bx0 + rv[3][k, cs] * bx1
                        outv[k, cs] = t0 * by0 + t1 * by1
                return carry2

            lax.fori_loop(0, _K // _UNROLL, px_body, 0)

        def awrite(ci, outv, sem):
            pltpu.async_copy(outv, out_hbm.at[pl.ds(tbase + ci * _K, _K)], sem)

        def waitout(outv, sem):
            pltpu.make_async_copy(out_hbm.at[pl.ds(tbase, _K)], outv, sem).wait()

        lc(0, ia, wa)
        launch(ia, ra, sema)
        lc(1, ib, wb)
        launch(ib, rb, semb)
        wait4(ia, ra, sema)
        combine(wa, ra, outa)
        awrite(0, outa, semoa)
        lc(2, ia, wa)
        launch(ia, ra, sema)
        wait4(ib, rb, semb)
        combine(wb, rb, outb)
        awrite(1, outb, semob)

        def body(p, carry):
            lc(2 * p + 1, ib, wb)
            launch(ib, rb, semb)
            wait4(ia, ra, sema)
            waitout(outa, semoa)
            combine(wa, ra, outa)
            awrite(2 * p, outa, semoa)
            lc(2 * p + 2, ia, wa)
            launch(ia, ra, sema)
            wait4(ib, rb, semb)
            waitout(outb, semob)
            combine(wb, rb, outb)
            awrite(2 * p + 1, outb, semob)
            return carry

        lax.fori_loop(1, npairs - 1, body, 0)

        lc(nchunk - 1, ib, wb)
        launch(ib, rb, semb)
        wait4(ia, ra, sema)
        waitout(outa, semoa)
        combine(wa, ra, outa)
        awrite(nchunk - 2, outa, semoa)
        wait4(ib, rb, semb)
        waitout(outb, semob)
        combine(wb, rb, outb)
        awrite(nchunk - 1, outb, semob)
        waitout(outa, semoa)
        waitout(outb, semob)

    return kern(table, gx, gy)


def kernel(inputImages, grids):
    c, h, w = inputImages.shape
    b, hg, wg, _ = grids.shape
    npix = b * hg * wg
    table = inputImages.reshape(c, h * w).T
    gx = grids[..., 0].reshape(-1)
    gy = grids[..., 1].reshape(-1)
    out_t = _bilinear_sc(table, gx, gy, npix, h, w, c)
    return out_t.reshape(b, hg, wg, c).transpose(0, 3, 1, 2)
